# Initial kernel scaffold; baseline (speedup 1.0000x reference)
import jax
import jax.numpy as jnp
from jax.experimental import pallas as pl


def kernel(x, W1, b1, W2, b2, W3, b3, Ws, bs, Wh, bh, Wo, bo):
    raise NotImplementedError("write your pallas kernel here")


# speedup vs baseline: 4.7537x; 4.7537x over previous
import functools

import jax
import jax.numpy as jnp
import numpy as np
from jax import lax
from jax.experimental import pallas as pl
from jax.experimental.pallas import tpu as pltpu
from jax.experimental.pallas import tpu_sc as plsc

N = 10000
NPAD = 10240
D_IN = 256
HID = 256
OUT = 256
SPACE = 4
SPAD = 128
PROP = 64
K = 16


RB_MLP = 2000


def _mlp_body(x_ref, w1_ref, b1_ref, w2_ref, b2_ref, w3_ref, b3_ref,
              ws_ref, bs_ref, wh_ref, bh_ref, d_ref, s_ref, h_ref):
    x = x_ref[...]
    t = jnp.tanh(jnp.dot(x, w1_ref[...], preferred_element_type=jnp.float32)
                 + b1_ref[...])
    t = jnp.tanh(jnp.dot(t, w2_ref[...], preferred_element_type=jnp.float32)
                 + b2_ref[...])
    d = jnp.dot(t, w3_ref[...], preferred_element_type=jnp.float32) + b3_ref[...]
    d_ref[...] = d
    s_ref[...] = (jnp.dot(d, ws_ref[...], preferred_element_type=jnp.float32)
                  + bs_ref[...])
    h_ref[...] = (jnp.dot(d, wh_ref[...], preferred_element_type=jnp.float32)
                  + bh_ref[...])


def _mlp(x, W1, b1, W2, b2, W3, b3, Ws_pad, bs_pad, Wh, bh):
    g = N // RB_MLP
    full = lambda shape: pl.BlockSpec(shape, lambda i: (0, 0))
    row = lambda w: pl.BlockSpec((RB_MLP, w), lambda i: (i, 0))
    return pl.pallas_call(
        _mlp_body,
        grid=(g,),
        in_specs=[row(D_IN), full((D_IN, HID)), full((1, HID)),
                  full((HID, HID)), full((1, HID)),
                  full((HID, HID)), full((1, HID)),
                  full((HID, SPAD)), full((1, SPAD)),
                  full((HID, PROP)), full((1, PROP))],
        out_specs=[row(HID), row(SPAD), row(PROP)],
        out_shape=[jax.ShapeDtypeStruct((N, HID), jnp.float32),
                   jax.ShapeDtypeStruct((N, SPAD), jnp.float32),
                   jax.ShapeDtypeStruct((N, PROP), jnp.float32)],
    )(x, W1, b1, W2, b2, W3, b3, Ws_pad, bs_pad, Wh, bh)


QB = 256


def _knn_body(q_ref, st_ref, idx_ref, w_ref):
    q = q_ref[...]
    st = st_ref[...]
    q2 = jnp.sum(q * q, axis=1, keepdims=True)
    s2 = jnp.sum(st * st, axis=0, keepdims=True)
    dm = (q2 - 2.0 * jnp.dot(q, st, preferred_element_type=jnp.float32)) + s2
    col = lax.broadcasted_iota(jnp.int32, (QB, NPAD), 1)
    idx_cols = []
    w_cols = []
    for _ in range(K):
        m = jnp.min(dm, axis=1, keepdims=True)
        cand = jnp.where(dm == m, col, NPAD)
        j = jnp.min(cand, axis=1, keepdims=True)
        idx_cols.append(j)
        w_cols.append(jnp.exp(-10.0 * jnp.maximum(m, 0.0)))
        dm = jnp.where(col == j, jnp.float32(np.inf), dm)
    idx_ref[...] = jnp.concatenate(idx_cols, axis=1)
    w_ref[...] = jnp.concatenate(w_cols, axis=1)


def _knn(s_pad, st):
    g = NPAD // QB
    return pl.pallas_call(
        _knn_body,
        grid=(g,),
        in_specs=[pl.BlockSpec((QB, SPAD), lambda i: (i, 0)),
                  pl.BlockSpec((SPAD, NPAD), lambda i: (0, 0))],
        out_specs=[pl.BlockSpec((QB, K), lambda i: (i, 0)),
                   pl.BlockSpec((QB, K), lambda i: (i, 0))],
        out_shape=[jax.ShapeDtypeStruct((NPAD, K), jnp.int32),
                   jax.ShapeDtypeStruct((NPAD, K), jnp.float32)],
    )(s_pad, st)


SC_NC = 2
SC_NS = 16
SC_NW = SC_NC * SC_NS
PW = NPAD // SC_NW
CH = 32
NCH = PW // CH
E = CH * K
GW = 128
NG = E // GW
HPAD = 128


def _sc_gather_body(h_hbm, idxf_hbm, w_hbm, mean_hbm, max_hbm,
                    idx_v, w_v, rows_v, mean_v, max_v, sem):
    wid = lax.axis_index("s") * SC_NC + lax.axis_index("c")
    base = wid * PW
    inv_k = jnp.float32(1.0 / K)

    for c in range(NCH):
        nb = base + c * CH
        pltpu.sync_copy(idxf_hbm.at[pl.ds(nb * K, E)], idx_v)
        pltpu.sync_copy(w_hbm.at[pl.ds(nb, CH)], w_v)
        cps = []
        for gidx in range(NG):
            cps.append(pltpu.async_copy(
                h_hbm.at[idx_v.at[pl.ds(gidx * GW, GW)]],
                rows_v.at[pl.ds(gidx * GW, GW)], sem))
        for cp in cps:
            cp.wait()

        def node_body(n, _):
            acc_s = []
            acc_m = []
            for j in range(PROP // 16):
                acc_s.append(jnp.zeros((16,), jnp.float32))
                acc_m.append(jnp.full((16,), -np.inf, jnp.float32))
            w_row = w_v[n, :]
            for k in range(K):
                wk = w_row[k]
                e = n * K + k
                for j in range(PROP // 16):
                    msg = rows_v[e, pl.ds(j * 16, 16)] * wk
                    acc_s[j] = acc_s[j] + msg
                    acc_m[j] = jnp.maximum(acc_m[j], msg)
            for j in range(PROP // 16):
                mean_v[n, pl.ds(j * 16, 16)] = acc_s[j] * inv_k
                max_v[n, pl.ds(j * 16, 16)] = acc_m[j]
            return 0

        lax.fori_loop(0, CH, node_body, 0)
        pltpu.sync_copy(mean_v, mean_hbm.at[pl.ds(nb, CH)])
        pltpu.sync_copy(max_v, max_hbm.at[pl.ds(nb, CH)])


def _sc_gather(h_pad, idx_flat, w):
    mesh = plsc.VectorSubcoreMesh(core_axis_name="c", subcore_axis_name="s",
                                  num_cores=SC_NC, num_subcores=SC_NS)
    fn = pl.kernel(
        _sc_gather_body,
        out_type=[jax.ShapeDtypeStruct((NPAD, HPAD), jnp.float32),
                  jax.ShapeDtypeStruct((NPAD, HPAD), jnp.float32)],
        mesh=mesh,
        scratch_types=[pltpu.VMEM((E,), jnp.int32),
                       pltpu.VMEM((CH, K), jnp.float32),
                       pltpu.VMEM((E, HPAD), jnp.float32),
                       pltpu.VMEM((CH, HPAD), jnp.float32),
                       pltpu.VMEM((CH, HPAD), jnp.float32),
                       pltpu.SemaphoreType.DMA],
    )
    return fn(h_pad, idx_flat, w)


RB_OUT = 2000


def _out_body(d_ref, mn_ref, mx_ref, wd_ref, wm_ref, wx_ref, bo_ref, o_ref):
    o = jnp.dot(d_ref[...], wd_ref[...], preferred_element_type=jnp.float32)
    o += jnp.dot(mn_ref[...], wm_ref[...], preferred_element_type=jnp.float32)
    o += jnp.dot(mx_ref[...], wx_ref[...], preferred_element_type=jnp.float32)
    o_ref[...] = o + bo_ref[...]


def _outmm(d, mn, mx, Wo_d, Wo_m, Wo_x, bo):
    g = N // RB_OUT
    full = lambda shape: pl.BlockSpec(shape, lambda i: (0, 0))
    row = lambda w: pl.BlockSpec((RB_OUT, w), lambda i: (i, 0))
    return pl.pallas_call(
        _out_body,
        grid=(g,),
        in_specs=[row(HID), row(PROP), row(PROP),
                  full((HID, OUT)), full((PROP, OUT)), full((PROP, OUT)),
                  full((1, OUT))],
        out_specs=row(OUT),
        out_shape=jax.ShapeDtypeStruct((N, OUT), jnp.float32),
    )(d, mn, mx, Wo_d, Wo_m, Wo_x, bo)


def kernel(x, W1, b1, W2, b2, W3, b3, Ws, bs, Wh, bh, Wo, bo):
    f32 = jnp.float32
    Ws_pad = jnp.zeros((HID, SPAD), f32).at[:, :SPACE].set(Ws)
    bs_pad = jnp.zeros((1, SPAD), f32).at[:, :SPACE].set(bs[None, :])

    d, s, h = _mlp(x, W1, b1[None, :], W2, b2[None, :], W3, b3[None, :],
                   Ws_pad, bs_pad, Wh, bh[None, :])

    s_pad = jnp.concatenate(
        [s, jnp.full((NPAD - N, SPAD), 1e6, f32)], axis=0)
    st = s_pad.T
    idx, w = _knn(s_pad, st)

    h_pad = jnp.pad(h, ((0, NPAD - N), (0, HPAD - PROP)))
    mean_pad, max_pad = _sc_gather(h_pad, idx.reshape(-1), w)

    return _outmm(d, mean_pad[:N, :PROP], max_pad[:N, :PROP],
                  Wo[:HID], Wo[HID:HID + PROP], Wo[HID + PROP:], bo[None, :])
